# baseline (speedup 1.0000x reference)
import functools

import jax
import jax.numpy as jnp
from jax import lax
from jax.experimental import pallas as pl
from jax.experimental.pallas import tpu as pltpu
from jax.experimental.pallas import tpu_sc as plsc

N = 10000
DEG = 32
D = 128
DW = D // 2
LN_EPS = 1e-5

NW = 32
NPAD = 10240
NODES_PER_W = NPAD // NW
G_NODES = 2
G_ROWS = G_NODES * DEG
NBUF = 4
STEPS = NODES_PER_W // (G_NODES * NBUF)

_MASK_HI = jnp.int32(-65536)


def _sc_gather_sum(xp, eidx_flat):
    mesh = plsc.VectorSubcoreMesh(core_axis_name="c", subcore_axis_name="s")
    info = plsc.get_sparse_core_info()
    nc = info.num_cores

    @functools.partial(
        pl.kernel,
        mesh=mesh,
        compiler_params=pltpu.CompilerParams(use_tc_tiling_on_sc=False),
        out_type=jax.ShapeDtypeStruct((NPAD, D), jnp.float32),
        scratch_types=[
            pltpu.VMEM((NODES_PER_W * DEG,), jnp.int32),
            pltpu.VMEM((NBUF, G_ROWS, DW), jnp.int32),
            pltpu.VMEM((NBUF * G_NODES, D), jnp.float32),
            pltpu.VMEM_SHARED((N, DW), jnp.int32),
            pltpu.SemaphoreType.DMA,
            pltpu.SemaphoreType.DMA,
            pltpu.SemaphoreType.DMA,
            pltpu.SemaphoreType.DMA,
        ],
    )
    def k(xp_hbm, idx_hbm, out_hbm, idx_v, rows_v, acc_v, xs_v, *sems):
        wid = lax.axis_index("s") * nc + lax.axis_index("c")
        sid = lax.axis_index("s")
        ibase = wid * (NODES_PER_W * DEG)
        srows = N // 16
        pltpu.sync_copy(xp_hbm.at[pl.ds(sid * srows, srows)],
                        xs_v.at[pl.ds(sid * srows, srows)])
        pltpu.sync_copy(idx_hbm.at[pl.ds(ibase, NODES_PER_W * DEG)], idx_v)
        plsc.subcore_barrier()

        def fire(g, b):
            off = g * G_ROWS
            pltpu.make_async_copy(
                xs_v.at[idx_v.at[pl.ds(off, G_ROWS)]],
                rows_v.at[b], sems[b]).start()

        for b in range(NBUF):
            fire(b, b)

        def body(t, carry):
            for b in range(NBUF):
                pltpu.make_async_copy(
                    xs_v.at[idx_v.at[pl.ds(0, G_ROWS)]],
                    rows_v.at[b], sems[b]).wait()
                for node in range(G_NODES):
                    base = node * DEG
                    for c in range(DW // 16):
                        sl = pl.ds(c * 16, 16)
                        w0 = rows_v[b, base, sl]
                        alo = lax.bitcast_convert_type(
                            lax.shift_left(w0, 16), jnp.float32)
                        ahi = lax.bitcast_convert_type(
                            lax.bitwise_and(w0, _MASK_HI), jnp.float32)
                        for r in range(1, DEG):
                            w = rows_v[b, base + r, sl]
                            alo = alo + lax.bitcast_convert_type(
                                lax.shift_left(w, 16), jnp.float32)
                            ahi = ahi + lax.bitcast_convert_type(
                                lax.bitwise_and(w, _MASK_HI), jnp.float32)
                        row = b * G_NODES + node
                        acc_v[row, pl.ds(16 * c, 16)] = alo
                        acc_v[row, pl.ds(64 + 16 * c, 16)] = ahi

                @pl.when(t < STEPS - 1)
                def _():
                    fire(t * NBUF + b + NBUF, b)

            rbase = wid * NODES_PER_W + t * (NBUF * G_NODES)
            pltpu.sync_copy(acc_v,
                            out_hbm.at[pl.ds(rbase, NBUF * G_NODES)])
            return carry

        lax.fori_loop(0, STEPS, body, 0)

    return k(xp, eidx_flat)


def _tc_mlp(x, neigh, eps, W, b, gamma, beta):
    BLK = 1000
    grid = (N // BLK,)

    def body(eps_ref, x_ref, ng_ref, w_ref, b_ref, g_ref, be_ref, o_ref):
        scale = 1.0 + eps_ref[0, 0]
        h = scale * x_ref[...] + ng_ref[...]
        y = lax.dot_general(h, w_ref[...], (((1,), (1,)), ((), ())),
                            preferred_element_type=jnp.float32) + b_ref[...]
        mu = jnp.mean(y, axis=-1, keepdims=True)
        var = jnp.mean((y - mu) ** 2, axis=-1, keepdims=True)
        o_ref[...] = (y - mu) * lax.rsqrt(var + LN_EPS) * g_ref[...] + be_ref[...]

    return pl.pallas_call(
        body,
        grid=grid,
        in_specs=[
            pl.BlockSpec((1, 1), lambda i: (0, 0)),
            pl.BlockSpec((BLK, D), lambda i: (i, 0)),
            pl.BlockSpec((BLK, D), lambda i: (i, 0)),
            pl.BlockSpec((D, D), lambda i: (0, 0)),
            pl.BlockSpec((1, D), lambda i: (0, 0)),
            pl.BlockSpec((1, D), lambda i: (0, 0)),
            pl.BlockSpec((1, D), lambda i: (0, 0)),
        ],
        out_specs=pl.BlockSpec((BLK, D), lambda i: (i, 0)),
        out_shape=jax.ShapeDtypeStruct((N, D), jnp.float32),
    )(eps.reshape(1, 1), x, neigh, W, b.reshape(1, D), gamma.reshape(1, D),
      beta.reshape(1, D))


def kernel(x, edge_index, eps, W, b, gamma, beta):
    eidx = jnp.pad(edge_index, ((0, NPAD - N), (0, 0))).reshape(-1)
    u = lax.bitcast_convert_type(x.astype(jnp.bfloat16), jnp.uint16)
    lo = u[:, :DW].astype(jnp.int32)
    hi = u[:, DW:].astype(jnp.int32)
    xp = lax.bitwise_or(lo, lax.shift_left(hi, 16))
    neigh = _sc_gather_sum(xp, eidx)
    return _tc_mlp(x, neigh, eps, W, b, gamma, beta)


# speedup vs baseline: 1.0280x; 1.0280x over previous
import functools

import jax
import jax.numpy as jnp
from jax import lax
from jax.experimental import pallas as pl
from jax.experimental.pallas import tpu as pltpu
from jax.experimental.pallas import tpu_sc as plsc

N = 10000
DEG = 32
D = 128
DW = D // 2
LN_EPS = 1e-5

NW = 32
NPAD = 10240
NODES_PER_W = NPAD // NW
G_NODES = 2
G_ROWS = G_NODES * DEG
NBUF = 4
STEPS = NODES_PER_W // (G_NODES * NBUF)

_MASK_HI = jnp.int32(-65536)


def _sc_gather_sum(xp, eidx_flat):
    mesh = plsc.VectorSubcoreMesh(core_axis_name="c", subcore_axis_name="s")
    info = plsc.get_sparse_core_info()
    nc = info.num_cores

    @functools.partial(
        pl.kernel,
        mesh=mesh,
        compiler_params=pltpu.CompilerParams(use_tc_tiling_on_sc=False),
        out_type=jax.ShapeDtypeStruct((NPAD, D), jnp.float32),
        scratch_types=[
            pltpu.VMEM((NODES_PER_W * DEG,), jnp.int32),
            pltpu.VMEM((NBUF, G_ROWS, DW), jnp.int32),
            pltpu.VMEM((NBUF * G_NODES, D), jnp.float32),
            pltpu.VMEM_SHARED((N, DW), jnp.int32),
            pltpu.SemaphoreType.DMA,
            pltpu.SemaphoreType.DMA,
            pltpu.SemaphoreType.DMA,
            pltpu.SemaphoreType.DMA,
        ],
    )
    def k(xp_hbm, idx_hbm, out_hbm, idx_v, rows_v, acc_v, xs_v, *sems):
        wid = lax.axis_index("s") * nc + lax.axis_index("c")
        sid = lax.axis_index("s")
        ibase = wid * (NODES_PER_W * DEG)
        srows = N // 16
        pltpu.sync_copy(xp_hbm.at[pl.ds(sid * srows, srows)],
                        xs_v.at[pl.ds(sid * srows, srows)])
        pltpu.sync_copy(idx_hbm.at[pl.ds(ibase, NODES_PER_W * DEG)], idx_v)
        plsc.subcore_barrier()

        def fire(g, b):
            off = g * G_ROWS
            pltpu.make_async_copy(
                xs_v.at[idx_v.at[pl.ds(off, G_ROWS)]],
                rows_v.at[b], sems[b]).start()

        for b in range(NBUF):
            fire(b, b)

        def body(t, carry):
            for b in range(NBUF):
                pltpu.make_async_copy(
                    xs_v.at[idx_v.at[pl.ds(0, G_ROWS)]],
                    rows_v.at[b], sems[b]).wait()
                for node in range(G_NODES):
                    base = node * DEG
                    for c in range(DW // 16):
                        sl = pl.ds(c * 16, 16)
                        w0 = rows_v[b, base, sl]
                        alo = lax.bitcast_convert_type(
                            lax.shift_left(w0, 16), jnp.float32)
                        ahi = lax.bitcast_convert_type(
                            lax.bitwise_and(w0, _MASK_HI), jnp.float32)
                        for r in range(1, DEG):
                            w = rows_v[b, base + r, sl]
                            alo = alo + lax.bitcast_convert_type(
                                lax.shift_left(w, 16), jnp.float32)
                            ahi = ahi + lax.bitcast_convert_type(
                                lax.bitwise_and(w, _MASK_HI), jnp.float32)
                        row = b * G_NODES + node
                        acc_v[row, pl.ds(16 * c, 16)] = alo
                        acc_v[row, pl.ds(64 + 16 * c, 16)] = ahi

                @pl.when(t < STEPS - 1)
                def _():
                    fire(t * NBUF + b + NBUF, b)

            rbase = wid * NODES_PER_W + t * (NBUF * G_NODES)
            pltpu.sync_copy(acc_v,
                            out_hbm.at[pl.ds(rbase, NBUF * G_NODES)])
            return carry

        lax.fori_loop(0, STEPS, body, 0)

    return k(xp, eidx_flat)


def _tc_mlp(x, neigh, eps, W, b, gamma, beta):
    BLK = 5000
    grid = (N // BLK,)

    def body(eps_ref, x_ref, ng_ref, w_ref, b_ref, g_ref, be_ref, o_ref):
        scale = 1.0 + eps_ref[0, 0]
        h = scale * x_ref[...] + ng_ref[...]
        y = lax.dot_general(h, w_ref[...], (((1,), (1,)), ((), ())),
                            preferred_element_type=jnp.float32) + b_ref[...]
        mu = jnp.mean(y, axis=-1, keepdims=True)
        var = jnp.mean((y - mu) ** 2, axis=-1, keepdims=True)
        o_ref[...] = (y - mu) * lax.rsqrt(var + LN_EPS) * g_ref[...] + be_ref[...]

    return pl.pallas_call(
        body,
        grid=grid,
        in_specs=[
            pl.BlockSpec((1, 1), lambda i: (0, 0)),
            pl.BlockSpec((BLK, D), lambda i: (i, 0)),
            pl.BlockSpec((BLK, D), lambda i: (i, 0)),
            pl.BlockSpec((D, D), lambda i: (0, 0)),
            pl.BlockSpec((1, D), lambda i: (0, 0)),
            pl.BlockSpec((1, D), lambda i: (0, 0)),
            pl.BlockSpec((1, D), lambda i: (0, 0)),
        ],
        out_specs=pl.BlockSpec((BLK, D), lambda i: (i, 0)),
        out_shape=jax.ShapeDtypeStruct((N, D), jnp.float32),
    )(eps.reshape(1, 1), x, neigh, W, b.reshape(1, D), gamma.reshape(1, D),
      beta.reshape(1, D))


def kernel(x, edge_index, eps, W, b, gamma, beta):
    eidx = jnp.pad(edge_index, ((0, NPAD - N), (0, 0))).reshape(-1)
    u = lax.bitcast_convert_type(x.astype(jnp.bfloat16), jnp.uint16)
    lo = u[:, :DW].astype(jnp.int32)
    hi = u[:, DW:].astype(jnp.int32)
    xp = lax.bitwise_or(lo, lax.shift_left(hi, 16))
    neigh = _sc_gather_sum(xp, eidx)
    return _tc_mlp(x, neigh, eps, W, b, gamma, beta)
